# baseline (device time: 21365 ns/iter reference)
import jax
import jax.numpy as jnp
from jax import lax
from jax.experimental import pallas as pl
from jax.experimental.pallas import tpu as pltpu

N_DEV = 16
HALVES = 2


def kernel(t):
    m, n = t.shape
    mc = m // N_DEV
    mh = mc // HALVES

    def body(x_ref, out_ref, recv_ref, res_ref,
             send1_sems, recv1_sems, send2_sems, recv2_sems):
        my = lax.axis_index("i")

        barrier_sem = pltpu.get_barrier_semaphore()
        for j in range(N_DEV):
            @pl.when(my != j)
            def _():
                pl.semaphore_signal(
                    barrier_sem, inc=1,
                    device_id=(j,), device_id_type=pl.DeviceIdType.MESH,
                )
        pl.semaphore_wait(barrier_sem, N_DEV - 1)

        for h in range(HALVES):
            for j in range(N_DEV):
                @pl.when(my != j)
                def _():
                    rdma = pltpu.make_async_remote_copy(
                        src_ref=x_ref.at[pl.ds(j * mc + h * mh, mh)],
                        dst_ref=recv_ref.at[h, my],
                        send_sem=send1_sems.at[h, j],
                        recv_sem=recv1_sems.at[h, my],
                        device_id=(j,),
                        device_id_type=pl.DeviceIdType.MESH,
                    )
                    rdma.start()

        for h in range(HALVES):
            recv_ref[h, my] = x_ref[pl.ds(my * mc + h * mh, mh)]

        for h in range(HALVES):
            for k in range(N_DEV):
                @pl.when(my != k)
                def _():
                    recv = pltpu.make_async_remote_copy(
                        src_ref=x_ref.at[pl.ds(0, mh)],
                        dst_ref=recv_ref.at[h, k],
                        send_sem=send1_sems.at[h, k],
                        recv_sem=recv1_sems.at[h, k],
                        device_id=(k,),
                        device_id_type=pl.DeviceIdType.MESH,
                    )
                    recv.wait_recv()

            sv = jnp.sum(recv_ref[h], axis=0)
            r = jnp.maximum(sv, 0.0)
            res_ref[h] = jnp.tanh(sv) * sv * sv + r * r * r

            for j in range(N_DEV):
                @pl.when(my != j)
                def _():
                    rdma = pltpu.make_async_remote_copy(
                        src_ref=res_ref.at[h],
                        dst_ref=out_ref.at[pl.ds(my * mc + h * mh, mh)],
                        send_sem=send2_sems.at[h, j],
                        recv_sem=recv2_sems.at[h, my],
                        device_id=(j,),
                        device_id_type=pl.DeviceIdType.MESH,
                    )
                    rdma.start()

            out_ref[pl.ds(my * mc + h * mh, mh)] = res_ref[h]

        for h in range(HALVES):
            for k in range(N_DEV):
                @pl.when(my != k)
                def _():
                    recv = pltpu.make_async_remote_copy(
                        src_ref=res_ref.at[h],
                        dst_ref=out_ref.at[pl.ds(k * mc + h * mh, mh)],
                        send_sem=send2_sems.at[h, k],
                        recv_sem=recv2_sems.at[h, k],
                        device_id=(k,),
                        device_id_type=pl.DeviceIdType.MESH,
                    )
                    recv.wait_recv()

        for h in range(HALVES):
            for j in range(N_DEV):
                @pl.when(my != j)
                def _():
                    s1 = pltpu.make_async_remote_copy(
                        src_ref=x_ref.at[pl.ds(j * mc + h * mh, mh)],
                        dst_ref=recv_ref.at[h, my],
                        send_sem=send1_sems.at[h, j],
                        recv_sem=recv1_sems.at[h, my],
                        device_id=(j,),
                        device_id_type=pl.DeviceIdType.MESH,
                    )
                    s1.wait_send()
                    s2 = pltpu.make_async_remote_copy(
                        src_ref=res_ref.at[h],
                        dst_ref=out_ref.at[pl.ds(0, mh)],
                        send_sem=send2_sems.at[h, j],
                        recv_sem=recv2_sems.at[h, my],
                        device_id=(j,),
                        device_id_type=pl.DeviceIdType.MESH,
                    )
                    s2.wait_send()

    return pl.pallas_call(
        body,
        out_shape=jax.ShapeDtypeStruct((m, n), jnp.float32),
        in_specs=[pl.BlockSpec(memory_space=pltpu.VMEM)],
        out_specs=pl.BlockSpec(memory_space=pltpu.VMEM),
        scratch_shapes=[
            pltpu.VMEM((HALVES, N_DEV, mh, n), jnp.float32),
            pltpu.VMEM((HALVES, mh, n), jnp.float32),
            pltpu.SemaphoreType.DMA((HALVES, N_DEV)),
            pltpu.SemaphoreType.DMA((HALVES, N_DEV)),
            pltpu.SemaphoreType.DMA((HALVES, N_DEV)),
            pltpu.SemaphoreType.DMA((HALVES, N_DEV)),
        ],
        compiler_params=pltpu.CompilerParams(collective_id=0),
    )(t)
